# baseline (speedup 1.0000x reference)
import jax
import jax.numpy as jnp
from jax import lax
from jax.experimental import pallas as pl
from jax.experimental.pallas import tpu as pltpu
from jax.experimental.pallas import tpu_sc as plsc

E, R, S = 100, 100000, 16384
NC, NS, L = 2, 16, 16
NW = NC * NS
H = 2
RH = R // H
CH = 4096
NCH = S // CH
TASKS = E * 2 * H
NT = (TASKS + NW - 1) // NW


def _body(a_hbm, b_hbm, srt_hbm, da_hbm, db_hbm, outa_hbm, outb_hbm,
          row0, row1, idx0, idx1, val0, val1,
          sin0, sin1, sout0, sout1, sidx0, sidx1, sval0, sval1):
    wid = lax.axis_index("s") * NC + lax.axis_index("c")
    rows, idxb, valb = (row0, row1), (idx0, idx1), (val0, val1)
    sins, souts = (sin0, sin1), (sout0, sout1)
    sidxs, svals = (sidx0, sidx1), (sval0, sval1)

    def parts(t):
        tid = (t * NW + wid) % TASKS
        return tid // (2 * H), (tid // H) % 2, tid % H

    def row_off(e, h):
        return pl.multiple_of(e * R + h * RH, 8)

    def start_in(t, p):
        e, arr, h = parts(t)
        off = row_off(e, h)

        @pl.when(arr == 0)
        def _():
            pltpu.async_copy(a_hbm.at[pl.ds(off, RH)], rows[p], sins[p])

        @pl.when(arr == 1)
        def _():
            pltpu.async_copy(b_hbm.at[pl.ds(off, RH)], rows[p], sins[p])

    def wait_in(p):
        pltpu.make_async_copy(a_hbm.at[pl.ds(0, RH)], rows[p], sins[p]).wait()

    def start_out(t, p):
        e, arr, h = parts(t)
        off = row_off(e, h)

        @pl.when(arr == 0)
        def _():
            pltpu.async_copy(rows[p], outa_hbm.at[pl.ds(off, RH)], souts[p])

        @pl.when(arr == 1)
        def _():
            pltpu.async_copy(rows[p], outb_hbm.at[pl.ds(off, RH)], souts[p])

    def wait_out(p):
        pltpu.make_async_copy(rows[p], outa_hbm.at[pl.ds(0, RH)],
                              souts[p]).wait()

    def prefetch_chunk(t, c):
        e, arr, _ = parts(t)
        ioff = pl.multiple_of(e * S + c * CH, 8)
        pltpu.async_copy(srt_hbm.at[pl.ds(ioff, CH)], idxb[c % 2],
                         sidxs[c % 2])

        @pl.when(arr == 0)
        def _():
            pltpu.async_copy(da_hbm.at[pl.ds(c * CH, CH)], valb[c % 2],
                             svals[c % 2])

        @pl.when(arr == 1)
        def _():
            pltpu.async_copy(db_hbm.at[pl.ds(c * CH, CH)], valb[c % 2],
                             svals[c % 2])

    def wait_chunk(c):
        pltpu.make_async_copy(srt_hbm.at[pl.ds(0, CH)], idxb[c % 2],
                              sidxs[c % 2]).wait()
        pltpu.make_async_copy(da_hbm.at[pl.ds(0, CH)], valb[c % 2],
                              svals[c % 2]).wait()

    def taskstep(t, p, prefetch_next):
        _, _, h = parts(t)
        base = h * RH
        rowbuf = rows[p]
        wait_in(p)
        for c in range(NCH):
            wait_chunk(c)
            if c + 1 < NCH:
                prefetch_chunk(t, c + 1)
            elif prefetch_next:
                prefetch_chunk(t + 1, 0)
            idxc, valc = idxb[c % 2], valb[c % 2]

            def _inner(i, carry, idxc=idxc, valc=valc):
                idx = idxc[pl.ds(i * L, L)]
                v = valc[pl.ds(i * L, L)]
                mask = (idx >= base) & (idx < base + RH)
                plsc.addupdate_scatter(rowbuf, [idx - base], v, mask=mask)
                return carry

            lax.fori_loop(0, CH // L, _inner, 0, unroll=4)
        start_out(t, p)
        wait_out(1 - p)
        if prefetch_next:
            start_in(t + 1, 1 - p)

    prefetch_chunk(0, 0)
    start_in(0, 0)
    pltpu.async_copy(a_hbm.at[pl.ds(0, RH)], rows[1], souts[1])

    def pair(j, carry):
        taskstep(2 * j, 0, True)
        taskstep(2 * j + 1, 1, True)
        return carry

    lax.fori_loop(0, (NT - 1) // 2, pair, 0)
    taskstep(NT - 1, 0, False)
    wait_out(0)


_scatter_update = pl.kernel(
    _body,
    out_type=[jax.ShapeDtypeStruct((E * R,), jnp.float32),
              jax.ShapeDtypeStruct((E * R,), jnp.float32)],
    mesh=plsc.VectorSubcoreMesh(core_axis_name="c", subcore_axis_name="s",
                                num_cores=NC, num_subcores=NS),
    scratch_types=[pltpu.VMEM((RH,), jnp.float32),
                   pltpu.VMEM((RH,), jnp.float32),
                   pltpu.VMEM((CH,), jnp.int32),
                   pltpu.VMEM((CH,), jnp.int32),
                   pltpu.VMEM((CH,), jnp.float32),
                   pltpu.VMEM((CH,), jnp.float32)]
                  + [pltpu.SemaphoreType.DMA] * 8,
    compiler_params=pltpu.CompilerParams(needs_layout_passes=False),
)


@jax.jit
def kernel(a, b, samples_regions, da, db):
    srt = samples_regions.T.reshape(E * S)
    na, nb = _scatter_update(a.reshape(E * R), b.reshape(E * R), srt, da, db)
    return na.reshape(E, R), nb.reshape(E, R)


# speedup vs baseline: 1.8415x; 1.8415x over previous
import jax
import jax.numpy as jnp
from jax import lax
from jax.experimental import pallas as pl
from jax.experimental.pallas import tpu as pltpu
from jax.experimental.pallas import tpu_sc as plsc

E, R, S = 100, 100000, 16384
NC, NS, L = 2, 16, 16
NW = NC * NS
SPLIT = 50048
HOFF = (0, SPLIT)
HLEN = (SPLIT, R - SPLIT)
VCH = 8192
PAIRS = E * 2
NP = (PAIRS + NW - 1) // NW
NT = 2 * NP


def _body(a_hbm, b_hbm, srt_hbm, da_hbm, db_hbm, outa_hbm, outb_hbm,
          row0, row1, idx_v, val_v, sin0, sin1, sout0, sout1, sidx):
    wid = lax.axis_index("s") * NC + lax.axis_index("c")
    rows = (row0, row1)
    sins, souts = (sin0, sin1), (sout0, sout1)

    def parts(t):
        u = ((t // 2) * NW + wid) % PAIRS
        return u // 2, u % 2

    def start_idx(t):
        e, _ = parts(t)
        pltpu.async_copy(srt_hbm.at[e], idx_v, sidx)

    def wait_idx():
        pltpu.make_async_copy(srt_hbm.at[0], idx_v, sidx).wait()

    def start_in(t, p):
        e, arr = parts(t)
        sl = pl.ds(HOFF[p], HLEN[p])

        @pl.when(arr == 0)
        def _():
            pltpu.async_copy(a_hbm.at[e].at[sl], rows[p], sins[p])

        @pl.when(arr == 1)
        def _():
            pltpu.async_copy(b_hbm.at[e].at[sl], rows[p], sins[p])

    def wait_in(p):
        pltpu.make_async_copy(a_hbm.at[0].at[pl.ds(HOFF[p], HLEN[p])],
                              rows[p], sins[p]).wait()

    def start_out(t, p):
        e, arr = parts(t)
        sl = pl.ds(HOFF[p], HLEN[p])

        @pl.when(arr == 0)
        def _():
            pltpu.async_copy(rows[p], outa_hbm.at[e].at[sl], souts[p])

        @pl.when(arr == 1)
        def _():
            pltpu.async_copy(rows[p], outb_hbm.at[e].at[sl], souts[p])

    def wait_out(p):
        pltpu.make_async_copy(rows[p], outa_hbm.at[0].at[pl.ds(HOFF[p],
                                                               HLEN[p])],
                              souts[p]).wait()

    def taskstep(t, p, prefetch_next):
        _, arr = parts(t)
        wait_in(p)
        if p == 0:
            wait_idx()
        rowbuf = rows[p]
        for c in range(S // VCH):
            @pl.when(arr == 0)
            def _():
                pltpu.sync_copy(da_hbm.at[pl.ds(c * VCH, VCH)], val_v)

            @pl.when(arr == 1)
            def _():
                pltpu.sync_copy(db_hbm.at[pl.ds(c * VCH, VCH)], val_v)

            def _inner(i, carry, c=c):
                idx = idx_v[pl.ds(c * VCH + i * L, L)]
                v = val_v[pl.ds(i * L, L)]
                if p == 0:
                    mask = idx < SPLIT
                    tgt = idx
                else:
                    mask = idx >= SPLIT
                    tgt = idx - SPLIT
                plsc.addupdate_scatter(rowbuf, [tgt], v, mask=mask)
                return carry

            lax.fori_loop(0, VCH // L, _inner, 0, unroll=4)
        start_out(t, p)
        wait_out(1 - p)
        if prefetch_next:
            start_in(t + 1, 1 - p)
            if p == 1:
                start_idx(t + 1)

    start_idx(0)
    start_in(0, 0)
    pltpu.async_copy(a_hbm.at[0].at[pl.ds(HOFF[1], HLEN[1])], rows[1],
                     souts[1])

    def pair(j, carry):
        taskstep(2 * j, 0, True)
        taskstep(2 * j + 1, 1, True)
        return carry

    lax.fori_loop(0, NP - 1, pair, 0)
    taskstep(NT - 2, 0, True)
    taskstep(NT - 1, 1, False)
    wait_out(1)


_scatter_update = pl.kernel(
    _body,
    out_type=[jax.ShapeDtypeStruct((E, R), jnp.float32),
              jax.ShapeDtypeStruct((E, R), jnp.float32)],
    mesh=plsc.VectorSubcoreMesh(core_axis_name="c", subcore_axis_name="s",
                                num_cores=NC, num_subcores=NS),
    scratch_types=[pltpu.VMEM((HLEN[0],), jnp.float32),
                   pltpu.VMEM((HLEN[1],), jnp.float32),
                   pltpu.VMEM((S,), jnp.int32),
                   pltpu.VMEM((VCH,), jnp.float32)]
                  + [pltpu.SemaphoreType.DMA] * 5,
    compiler_params=pltpu.CompilerParams(needs_layout_passes=False),
)


@jax.jit
def kernel(a, b, samples_regions, da, db):
    srt = samples_regions.T
    return tuple(_scatter_update(a, b, srt, da, db))


# speedup vs baseline: 2.2256x; 1.2086x over previous
import jax
import jax.numpy as jnp
from jax import lax
from jax.experimental import pallas as pl
from jax.experimental.pallas import tpu as pltpu
from jax.experimental.pallas import tpu_sc as plsc

E, R, S = 100, 100000, 16384
NC, NS, L = 2, 16, 16
NW = NC * NS
SPLIT = 50048
HOFF = (0, SPLIT)
HLEN = (SPLIT, R - SPLIT)
VCH = 8192
PAIRS = E * 2
NP = (PAIRS + NW - 1) // NW
NT = 2 * NP


def _body(a_hbm, b_hbm, srt_hbm, da_hbm, db_hbm, outa_hbm, outb_hbm,
          row0, row1, idx_v, val_v, sin0, sin1, sout0, sout1, sidx):
    wid = lax.axis_index("s") * NC + lax.axis_index("c")
    rows = (row0, row1)
    sins, souts = (sin0, sin1), (sout0, sout1)

    def parts(t):
        u = ((t // 2) * NW + wid) % PAIRS
        return u // 2, u % 2

    def start_idx(t):
        e, _ = parts(t)
        pltpu.async_copy(srt_hbm.at[e], idx_v, sidx)

    def wait_idx():
        pltpu.make_async_copy(srt_hbm.at[0], idx_v, sidx).wait()

    def start_in(t, p):
        e, arr = parts(t)
        sl = pl.ds(HOFF[p], HLEN[p])

        @pl.when(arr == 0)
        def _():
            pltpu.async_copy(a_hbm.at[e].at[sl], rows[p], sins[p])

        @pl.when(arr == 1)
        def _():
            pltpu.async_copy(b_hbm.at[e].at[sl], rows[p], sins[p])

    def wait_in(p):
        pltpu.make_async_copy(a_hbm.at[0].at[pl.ds(HOFF[p], HLEN[p])],
                              rows[p], sins[p]).wait()

    def start_out(t, p):
        e, arr = parts(t)
        sl = pl.ds(HOFF[p], HLEN[p])

        @pl.when(arr == 0)
        def _():
            pltpu.async_copy(rows[p], outa_hbm.at[e].at[sl], souts[p])

        @pl.when(arr == 1)
        def _():
            pltpu.async_copy(rows[p], outb_hbm.at[e].at[sl], souts[p])

    def wait_out(p):
        pltpu.make_async_copy(rows[p], outa_hbm.at[0].at[pl.ds(HOFF[p],
                                                               HLEN[p])],
                              souts[p]).wait()

    def taskstep(t, p, prefetch_next):
        _, arr = parts(t)
        wait_in(p)
        if p == 0:
            wait_idx()
        rowbuf = rows[p]
        for c in range(S // VCH):
            @pl.when(arr == 0)
            def _():
                pltpu.sync_copy(da_hbm.at[pl.ds(c * VCH, VCH)], val_v)

            @pl.when(arr == 1)
            def _():
                pltpu.sync_copy(db_hbm.at[pl.ds(c * VCH, VCH)], val_v)

            def _inner(i, carry, c=c):
                idx = idx_v[pl.ds(c * VCH + i * L, L)]
                v = val_v[pl.ds(i * L, L)]
                if p == 0:
                    mask = idx < SPLIT
                    tgt = idx
                else:
                    mask = idx >= SPLIT
                    tgt = idx - SPLIT
                plsc.addupdate_scatter(rowbuf, [tgt], v, mask=mask)
                return carry

            lax.fori_loop(0, VCH // L, _inner, 0, unroll=8)
            if c == 0:
                wait_out(1 - p)
                if prefetch_next:
                    start_in(t + 1, 1 - p)
        start_out(t, p)
        if prefetch_next and p == 1:
            start_idx(t + 1)

    start_idx(0)
    start_in(0, 0)
    pltpu.async_copy(a_hbm.at[0].at[pl.ds(HOFF[1], HLEN[1])], rows[1],
                     souts[1])

    def pair(j, carry):
        taskstep(2 * j, 0, True)
        taskstep(2 * j + 1, 1, True)
        return carry

    lax.fori_loop(0, NP - 1, pair, 0)
    taskstep(NT - 2, 0, True)
    taskstep(NT - 1, 1, False)
    wait_out(1)


_scatter_update = pl.kernel(
    _body,
    out_type=[jax.ShapeDtypeStruct((E, R), jnp.float32),
              jax.ShapeDtypeStruct((E, R), jnp.float32)],
    mesh=plsc.VectorSubcoreMesh(core_axis_name="c", subcore_axis_name="s",
                                num_cores=NC, num_subcores=NS),
    scratch_types=[pltpu.VMEM((HLEN[0],), jnp.float32),
                   pltpu.VMEM((HLEN[1],), jnp.float32),
                   pltpu.VMEM((S,), jnp.int32),
                   pltpu.VMEM((VCH,), jnp.float32)]
                  + [pltpu.SemaphoreType.DMA] * 5,
    compiler_params=pltpu.CompilerParams(needs_layout_passes=False),
)


@jax.jit
def kernel(a, b, samples_regions, da, db):
    srt = samples_regions.T
    return tuple(_scatter_update(a, b, srt, da, db))


# speedup vs baseline: 3.2028x; 1.4391x over previous
import jax
import jax.numpy as jnp
from jax import lax
from jax.experimental import pallas as pl
from jax.experimental.pallas import tpu as pltpu
from jax.experimental.pallas import tpu_sc as plsc

E, R, S = 100, 100000, 16384
NC, NS, L = 2, 16, 16
NW = NC * NS
SPLIT = 50048
HOFF = (0, SPLIT)
HLEN = (SPLIT, R - SPLIT)
VCH = 8192
PAIRS = E * 2
NP = (PAIRS + NW - 1) // NW
NT = 2 * NP


def _body(a_hbm, b_hbm, srt_hbm, da_hbm, db_hbm, outa_hbm, outb_hbm,
          row0, row1, idx_v, val_v, sin0, sin1, sout0, sout1, sidx):
    wid = lax.axis_index("s") * NC + lax.axis_index("c")
    rows = (row0, row1)
    sins, souts = (sin0, sin1), (sout0, sout1)

    def parts(t):
        u = ((t // 2) * NW + wid) % PAIRS
        return u // 2, u % 2

    def start_idx(t):
        e, _ = parts(t)
        pltpu.async_copy(srt_hbm.at[e], idx_v, sidx)

    def wait_idx():
        pltpu.make_async_copy(srt_hbm.at[0], idx_v, sidx).wait()

    def start_in(t, p):
        e, arr = parts(t)
        sl = pl.ds(HOFF[p], HLEN[p])

        @pl.when(arr == 0)
        def _():
            pltpu.async_copy(a_hbm.at[e].at[sl], rows[p], sins[p])

        @pl.when(arr == 1)
        def _():
            pltpu.async_copy(b_hbm.at[e].at[sl], rows[p], sins[p])

    def wait_in(p):
        pltpu.make_async_copy(a_hbm.at[0].at[pl.ds(HOFF[p], HLEN[p])],
                              rows[p], sins[p]).wait()

    def start_out(t, p):
        e, arr = parts(t)
        sl = pl.ds(HOFF[p], HLEN[p])

        @pl.when(arr == 0)
        def _():
            pltpu.async_copy(rows[p], outa_hbm.at[e].at[sl], souts[p])

        @pl.when(arr == 1)
        def _():
            pltpu.async_copy(rows[p], outb_hbm.at[e].at[sl], souts[p])

    def wait_out(p):
        pltpu.make_async_copy(rows[p], outa_hbm.at[0].at[pl.ds(HOFF[p],
                                                               HLEN[p])],
                              souts[p]).wait()

    def taskstep(t, p, prefetch_next):
        _, arr = parts(t)
        wait_in(p)
        if p == 0:
            wait_idx()
        rowbuf = rows[p]
        for c in range(S // VCH):
            @pl.when(arr == 0)
            def _():
                pltpu.sync_copy(da_hbm.at[pl.ds(c * VCH, VCH)], val_v)

            @pl.when(arr == 1)
            def _():
                pltpu.sync_copy(db_hbm.at[pl.ds(c * VCH, VCH)], val_v)

            @plsc.parallel_loop(0, VCH // L, unroll=8)
            def _inner(i, c=c):
                idx = idx_v[pl.ds(c * VCH + i * L, L)]
                v = val_v[pl.ds(i * L, L)]
                if p == 0:
                    mask = idx < SPLIT
                    tgt = idx
                else:
                    mask = idx >= SPLIT
                    tgt = idx - SPLIT
                plsc.addupdate_scatter(rowbuf, [tgt], v, mask=mask)
            if c == 0:
                wait_out(1 - p)
                if prefetch_next:
                    start_in(t + 1, 1 - p)
        start_out(t, p)
        if prefetch_next and p == 1:
            start_idx(t + 1)

    start_idx(0)
    start_in(0, 0)
    pltpu.async_copy(a_hbm.at[0].at[pl.ds(HOFF[1], HLEN[1])], rows[1],
                     souts[1])

    def pair(j, carry):
        taskstep(2 * j, 0, True)
        taskstep(2 * j + 1, 1, True)
        return carry

    lax.fori_loop(0, NP - 1, pair, 0)
    taskstep(NT - 2, 0, True)
    taskstep(NT - 1, 1, False)
    wait_out(1)


_scatter_update = pl.kernel(
    _body,
    out_type=[jax.ShapeDtypeStruct((E, R), jnp.float32),
              jax.ShapeDtypeStruct((E, R), jnp.float32)],
    mesh=plsc.VectorSubcoreMesh(core_axis_name="c", subcore_axis_name="s",
                                num_cores=NC, num_subcores=NS),
    scratch_types=[pltpu.VMEM((HLEN[0],), jnp.float32),
                   pltpu.VMEM((HLEN[1],), jnp.float32),
                   pltpu.VMEM((S,), jnp.int32),
                   pltpu.VMEM((VCH,), jnp.float32)]
                  + [pltpu.SemaphoreType.DMA] * 5,
    compiler_params=pltpu.CompilerParams(needs_layout_passes=False),
)


@jax.jit
def kernel(a, b, samples_regions, da, db):
    srt = samples_regions.T
    return tuple(_scatter_update(a, b, srt, da, db))


# speedup vs baseline: 3.2841x; 1.0254x over previous
import jax
import jax.numpy as jnp
from jax import lax
from jax.experimental import pallas as pl
from jax.experimental.pallas import tpu as pltpu
from jax.experimental.pallas import tpu_sc as plsc

E, R, S = 100, 100000, 16384
NC, NS, L = 2, 16, 16
NW = NC * NS
SPLIT = 50048
HOFF = (0, SPLIT)
HLEN = (SPLIT, R - SPLIT)
VCH = 4096
PAIRS = E * 2
NP = (PAIRS + NW - 1) // NW
NT = 2 * NP


def _body(a_hbm, b_hbm, srt_hbm, da_hbm, db_hbm, outa_hbm, outb_hbm,
          row0, row1, idx_v, val0, val1,
          sin0, sin1, sout0, sout1, sidx, sval0, sval1):
    wid = lax.axis_index("s") * NC + lax.axis_index("c")
    rows, valb = (row0, row1), (val0, val1)
    sins, souts = (sin0, sin1), (sout0, sout1)
    svals = (sval0, sval1)

    def start_val(t, c):
        _, arr = parts(t)

        @pl.when(arr == 0)
        def _():
            pltpu.async_copy(da_hbm.at[pl.ds(c * VCH, VCH)], valb[c % 2],
                             svals[c % 2])

        @pl.when(arr == 1)
        def _():
            pltpu.async_copy(db_hbm.at[pl.ds(c * VCH, VCH)], valb[c % 2],
                             svals[c % 2])

    def wait_val(c):
        pltpu.make_async_copy(da_hbm.at[pl.ds(0, VCH)], valb[c % 2],
                              svals[c % 2]).wait()

    def parts(t):
        u = ((t // 2) * NW + wid) % PAIRS
        return u // 2, u % 2

    def start_idx(t):
        e, _ = parts(t)
        pltpu.async_copy(srt_hbm.at[e], idx_v, sidx)

    def wait_idx():
        pltpu.make_async_copy(srt_hbm.at[0], idx_v, sidx).wait()

    def start_in(t, p):
        e, arr = parts(t)
        sl = pl.ds(HOFF[p], HLEN[p])

        @pl.when(arr == 0)
        def _():
            pltpu.async_copy(a_hbm.at[e].at[sl], rows[p], sins[p])

        @pl.when(arr == 1)
        def _():
            pltpu.async_copy(b_hbm.at[e].at[sl], rows[p], sins[p])

    def wait_in(p):
        pltpu.make_async_copy(a_hbm.at[0].at[pl.ds(HOFF[p], HLEN[p])],
                              rows[p], sins[p]).wait()

    def start_out(t, p):
        e, arr = parts(t)
        sl = pl.ds(HOFF[p], HLEN[p])

        @pl.when(arr == 0)
        def _():
            pltpu.async_copy(rows[p], outa_hbm.at[e].at[sl], souts[p])

        @pl.when(arr == 1)
        def _():
            pltpu.async_copy(rows[p], outb_hbm.at[e].at[sl], souts[p])

    def wait_out(p):
        pltpu.make_async_copy(rows[p], outa_hbm.at[0].at[pl.ds(HOFF[p],
                                                               HLEN[p])],
                              souts[p]).wait()

    def taskstep(t, p, prefetch_next):
        wait_in(p)
        if p == 0:
            wait_idx()
        rowbuf = rows[p]
        for c in range(S // VCH):
            wait_val(c)
            if c + 1 < S // VCH:
                start_val(t, c + 1)
            elif prefetch_next:
                start_val(t + 1, 0)
            valc = valb[c % 2]

            @plsc.parallel_loop(0, VCH // L, unroll=8)
            def _inner(i, c=c, valc=valc):
                idx = idx_v[pl.ds(c * VCH + i * L, L)]
                v = valc[pl.ds(i * L, L)]
                if p == 0:
                    mask = idx < SPLIT
                    tgt = idx
                else:
                    mask = idx >= SPLIT
                    tgt = idx - SPLIT
                plsc.addupdate_scatter(rowbuf, [tgt], v, mask=mask)
            if c == 1:
                wait_out(1 - p)
                if prefetch_next:
                    start_in(t + 1, 1 - p)
        start_out(t, p)
        if prefetch_next and p == 1:
            start_idx(t + 1)

    start_idx(0)
    start_val(0, 0)
    start_in(0, 0)
    pltpu.async_copy(a_hbm.at[0].at[pl.ds(HOFF[1], HLEN[1])], rows[1],
                     souts[1])

    def pair(j, carry):
        taskstep(2 * j, 0, True)
        taskstep(2 * j + 1, 1, True)
        return carry

    lax.fori_loop(0, NP - 1, pair, 0)
    taskstep(NT - 2, 0, True)
    taskstep(NT - 1, 1, False)
    wait_out(1)


_scatter_update = pl.kernel(
    _body,
    out_type=[jax.ShapeDtypeStruct((E, R), jnp.float32),
              jax.ShapeDtypeStruct((E, R), jnp.float32)],
    mesh=plsc.VectorSubcoreMesh(core_axis_name="c", subcore_axis_name="s",
                                num_cores=NC, num_subcores=NS),
    scratch_types=[pltpu.VMEM((HLEN[0],), jnp.float32),
                   pltpu.VMEM((HLEN[1],), jnp.float32),
                   pltpu.VMEM((S,), jnp.int32),
                   pltpu.VMEM((VCH,), jnp.float32),
                   pltpu.VMEM((VCH,), jnp.float32)]
                  + [pltpu.SemaphoreType.DMA] * 7,
    compiler_params=pltpu.CompilerParams(needs_layout_passes=False),
)


@jax.jit
def kernel(a, b, samples_regions, da, db):
    srt = samples_regions.T
    return tuple(_scatter_update(a, b, srt, da, db))
